# baseline (speedup 1.0000x reference)
import functools

import jax
import jax.numpy as jnp
from jax import lax
from jax.experimental import pallas as pl
from jax.experimental.pallas import tpu as pltpu
from jax.experimental.pallas import tpu_sc as plsc

SOURCE_SIZE = 1000000
COORDS = 64
CHANNELS = 32
N = 16384

NUM_CORES = 2
NUM_SUBCORES = 16
NW = NUM_CORES * NUM_SUBCORES
B_PER_W = N // NW
CHUNK = 128
NCHUNK = B_PER_W // CHUNK

_mesh = plsc.VectorSubcoreMesh(core_axis_name="c", subcore_axis_name="s")


@functools.partial(
    pl.kernel,
    mesh=_mesh,
    out_type=(
        jax.ShapeDtypeStruct((N, COORDS), jnp.float32),
        jax.ShapeDtypeStruct((N, CHANNELS), jnp.float32),
    ),
    scratch_types=[
        pltpu.VMEM((B_PER_W,), jnp.int32),
        pltpu.VMEM((B_PER_W, COORDS), jnp.float32),
        pltpu.VMEM((B_PER_W, CHANNELS), jnp.float32),
        pltpu.SemaphoreType.DMA,
        pltpu.SemaphoreType.DMA,
    ],
    compiler_params=pltpu.CompilerParams(use_tc_tiling_on_sc=False),
)
def _gather_kernel(x_hbm, y_hbm, idx_hbm, out_x, out_y,
                   idx_v, xrows, yrows, sem_x, sem_y):
    wid = lax.axis_index("s") * NUM_CORES + lax.axis_index("c")
    base = wid * B_PER_W
    pltpu.sync_copy(idx_hbm.at[pl.ds(base, B_PER_W)], idx_v)
    copies = []
    for j in range(NCHUNK):
        sl = pl.ds(j * CHUNK, CHUNK)
        copies.append(pltpu.async_copy(x_hbm.at[idx_v.at[sl]], xrows.at[sl], sem_x))
        copies.append(pltpu.async_copy(y_hbm.at[idx_v.at[sl]], yrows.at[sl], sem_y))
    for c in copies:
        c.wait()
    pltpu.sync_copy(xrows, out_x.at[pl.ds(base, B_PER_W)])
    pltpu.sync_copy(yrows, out_y.at[pl.ds(base, B_PER_W)])


def kernel(source_X, source_Y, indices):
    return _gather_kernel(source_X, source_Y, indices)


# speedup vs baseline: 1.3188x; 1.3188x over previous
import functools

import jax
import jax.numpy as jnp
from jax import lax
from jax.experimental import pallas as pl
from jax.experimental.pallas import tpu as pltpu
from jax.experimental.pallas import tpu_sc as plsc

SOURCE_SIZE = 1000000
COORDS = 64
CHANNELS = 32
N = 16384

NC, NS = 2, 16
NW = NC * NS
LANES = 16

NTILES = (SOURCE_SIZE + 127) // 128
NFULL = SOURCE_SIZE // 128
TAIL_BASE = NFULL * 128
TAIL_W = SOURCE_SIZE - TAIL_BASE
TPW = (NFULL + NW - 1) // NW
OUT_ROWS = N + NW
BATCH = 128

_mesh = plsc.VectorSubcoreMesh(core_axis_name="c", subcore_axis_name="s")


def _iota():
    return lax.iota(jnp.int32, LANES)


@functools.partial(
    pl.kernel,
    mesh=_mesh,
    out_type=jax.ShapeDtypeStruct((OUT_ROWS, 128), jnp.float32),
    scratch_types=[
        pltpu.VMEM((N + LANES,), jnp.int32),
        pltpu.VMEM((N,), jnp.int32),
        pltpu.VMEM((COORDS, 128), jnp.float32),
        pltpu.VMEM((CHANNELS, 128), jnp.float32),
        pltpu.VMEM((BATCH, 128), jnp.float32),
        pltpu.VMEM((BATCH,), jnp.int32),
        pltpu.SemaphoreType.DMA,
    ],
    compiler_params=pltpu.CompilerParams(
        use_tc_tiling_on_sc=True, needs_layout_passes=False),
)
def _scan_gather(xT_hbm, yT_hbm, xtail_hbm, ytail_hbm, idx_hbm, out_hbm,
                 idx_v, ent_p, stx, sty, obatch, obidx, sem):
    wid = lax.axis_index("s") * NC + lax.axis_index("c")
    lo = wid * TPW
    hi = jnp.minimum(lo + TPW, NFULL)
    is_last = wid == NW - 1
    fhi = jnp.where(is_last, NTILES, hi)
    dump = N + wid
    iota = _iota()

    pltpu.sync_copy(idx_hbm, idx_v.at[pl.ds(0, N)])

    def filt(k, cnt):
        iv = idx_v[pl.ds(pl.multiple_of(k * LANES, LANES), LANES)]
        t = lax.shift_right_logical(iv, 7)
        m = (t >= lo) & (t < fhi)
        mi = m.astype(jnp.int32)
        pos = cnt + plsc.cumsum(mi) - 1
        plsc.store_scatter(idx_v, [pos], iv, mask=m)
        plsc.store_scatter(ent_p, [pos], iota + k * LANES, mask=m)
        return cnt + jnp.sum(mi)

    cnt = lax.fori_loop(0, N // LANES, filt, jnp.int32(0))
    plsc.store_scatter(idx_v, [cnt + iota],
                       jnp.full((LANES,), -1, jnp.int32))
    nvec = (cnt + LANES - 1) // LANES

    for k in range(BATCH // LANES):
        obidx[pl.ds(k * LANES, LANES)] = jnp.full((LANES,), dump, jnp.int32)

    def extract_sample(l_s, p_s, slot):
        l_vec = jnp.zeros((LANES,), jnp.int32) + l_s
        slot_vec = jnp.zeros((LANES,), jnp.int32) + slot
        for k in range(COORDS // LANES):
            vals = plsc.load_gather(stx, [iota + k * LANES, l_vec])
            plsc.store_scatter(obatch, [slot_vec, iota + k * LANES], vals)
        for k in range(CHANNELS // LANES):
            vals = plsc.load_gather(sty, [iota + k * LANES, l_vec])
            plsc.store_scatter(obatch, [slot_vec, iota + (COORDS + k * LANES)],
                               vals)
        plsc.store_scatter(obidx, [slot_vec], jnp.zeros((LANES,), jnp.int32) + p_s,
                           mask=iota == 0)
        slot = slot + 1

        @pl.when(slot == BATCH)
        def _flush():
            pltpu.async_copy(obatch, out_hbm.at[obidx], sem).wait()
            for k in range(BATCH // LANES):
                obidx[pl.ds(k * LANES, LANES)] = jnp.full(
                    (LANES,), dump, jnp.int32)

        return jnp.where(slot == BATCH, 0, slot)

    def process_tile(t, slot):
        @pl.when(t < NFULL)
        def _full():
            off = pl.multiple_of(t * 128, 128)
            pltpu.sync_copy(xT_hbm.at[:, pl.ds(off, 128)], stx)
            pltpu.sync_copy(yT_hbm.at[:, pl.ds(off, 128)], sty)

        @pl.when(t == NFULL)
        def _tail():
            pltpu.sync_copy(xtail_hbm, stx)
            pltpu.sync_copy(ytail_hbm, sty)

        def scan_vec(j, slot):
            base = pl.multiple_of(j * LANES, LANES)
            e_i = idx_v[pl.ds(base, LANES)]
            e_p = ent_p[pl.ds(base, LANES)]
            m = lax.shift_right_logical(e_i, 7) == t

            def any_left(c):
                me, _ = c
                return jnp.any(me)

            def pull_one(c):
                me, slot = c
                ffs = plsc.all_reduce_ffs(me)
                sel = iota == ffs
                l_s = jnp.sum(jnp.where(sel, e_i & 127, 0))
                p_s = jnp.sum(jnp.where(sel, e_p, 0))
                slot = extract_sample(l_s, p_s, slot)
                return me & jnp.logical_not(sel), slot

            _, slot = lax.while_loop(any_left, pull_one, (m, slot))
            return slot

        return lax.fori_loop(0, nvec, scan_vec, slot)

    slot = lax.fori_loop(lo, fhi, process_tile, jnp.int32(0))
    pltpu.async_copy(obatch, out_hbm.at[obidx], sem).wait()
    del slot


def kernel(source_X, source_Y, indices):
    xtail = jnp.zeros((COORDS, 128), jnp.float32)
    xtail = xtail.at[:, :TAIL_W].set(source_X[TAIL_BASE:].T)
    ytail = jnp.zeros((CHANNELS, 128), jnp.float32)
    ytail = ytail.at[:, :TAIL_W].set(source_Y[TAIL_BASE:].T)
    out2 = _scan_gather(source_X.T, source_Y.T, xtail, ytail, indices)
    return out2[:N, :COORDS], out2[:N, COORDS:COORDS + CHANNELS]


# speedup vs baseline: 3.6208x; 2.7456x over previous
import functools

import jax
import jax.numpy as jnp
from jax import lax
from jax.experimental import pallas as pl
from jax.experimental.pallas import tpu as pltpu
from jax.experimental.pallas import tpu_sc as plsc

SOURCE_SIZE = 1000000
COORDS = 64
CHANNELS = 32
N = 16384

NC, NS = 2, 16
NW = NC * NS
LANES = 16

NTILES = (SOURCE_SIZE + 127) // 128
NFULL = SOURCE_SIZE // 128
TAIL_BASE = NFULL * 128
TAIL_W = SOURCE_SIZE - TAIL_BASE
GW = 256
TPW = 246
NSTEP = TPW // 2
OUT_ROWS = N + NW
BATCH = 128

_mesh = plsc.VectorSubcoreMesh(core_axis_name="c", subcore_axis_name="s")


@functools.partial(
    pl.kernel,
    mesh=_mesh,
    out_type=jax.ShapeDtypeStruct((OUT_ROWS, 128), jnp.float32),
    scratch_types=[
        pltpu.VMEM((N + LANES,), jnp.int32),
        pltpu.VMEM((N,), jnp.int32),
        pltpu.VMEM((COORDS, GW), jnp.float32),
        pltpu.VMEM((COORDS, GW), jnp.float32),
        pltpu.VMEM((CHANNELS, GW), jnp.float32),
        pltpu.VMEM((CHANNELS, GW), jnp.float32),
        pltpu.VMEM((BATCH, 128), jnp.float32),
        pltpu.VMEM((BATCH,), jnp.int32),
        pltpu.SemaphoreType.DMA,
        pltpu.SemaphoreType.DMA,
        pltpu.SemaphoreType.DMA,
    ],
    compiler_params=pltpu.CompilerParams(
        use_tc_tiling_on_sc=True, needs_layout_passes=False),
)
def _scan_gather(xT_hbm, yT_hbm, xtail_hbm, ytail_hbm, idx_hbm, out_hbm,
                 idx_v, ent_p, sx0, sx1, sy0, sy1, obatch, obidx,
                 sem0, sem1, ssem):
    wid = lax.axis_index("s") * NC + lax.axis_index("c")
    lo = wid * TPW
    hi = jnp.minimum(lo + TPW, NFULL)
    nstep = lax.max((hi - lo) // 2, 0)
    is_last = wid == NW - 1
    fhi = jnp.where(is_last, NTILES, hi)
    dump = N + wid
    iota = lax.iota(jnp.int32, LANES)
    bufs = ((sx0, sy0, sem0), (sx1, sy1, sem1))

    pltpu.sync_copy(idx_hbm, idx_v.at[pl.ds(0, N)])

    def filt(k, cnt):
        iv = idx_v[pl.ds(pl.multiple_of(k * LANES, LANES), LANES)]
        t = lax.shift_right_logical(iv, 7)
        m = (t >= lo) & (t < fhi)
        mi = m.astype(jnp.int32)
        pos = cnt + plsc.cumsum(mi) - 1
        plsc.store_scatter(idx_v, [pos], iv, mask=m)
        plsc.store_scatter(ent_p, [pos], iota + k * LANES, mask=m)
        return cnt + jnp.sum(mi)

    cnt = lax.fori_loop(0, N // LANES, filt, jnp.int32(0))
    plsc.store_scatter(idx_v, [cnt + iota],
                       jnp.full((LANES,), -1, jnp.int32))
    nvec = (cnt + LANES - 1) // LANES

    for k in range(BATCH // LANES):
        obidx[pl.ds(k * LANES, LANES)] = jnp.full((LANES,), dump, jnp.int32)

    def issue(r, bufi):
        sx, sy, sem = bufs[bufi]
        off = pl.multiple_of((lo + 2 * r) * 128, GW)
        pltpu.async_copy(xT_hbm.at[:, pl.ds(off, GW)], sx, sem)
        pltpu.async_copy(yT_hbm.at[:, pl.ds(off, GW)], sy, sem)

    def wait_fetch(bufi):
        sx, sy, sem = bufs[bufi]
        pltpu.make_async_copy(xT_hbm.at[:, pl.ds(0, GW)], sx, sem).wait()
        pltpu.make_async_copy(yT_hbm.at[:, pl.ds(0, GW)], sy, sem).wait()

    def extract_sample(sx, sy, e_i, e_p, sel, slot):
        l_s = jnp.sum(jnp.where(sel, e_i & (GW - 1), 0))
        p_s = jnp.sum(jnp.where(sel, e_p, 0))
        l_vec = jnp.zeros((LANES,), jnp.int32) + l_s
        slot_vec = jnp.zeros((LANES,), jnp.int32) + slot
        for k in range(COORDS // LANES):
            vals = plsc.load_gather(sx, [iota + k * LANES, l_vec])
            plsc.store_scatter(obatch, [slot_vec, iota + k * LANES], vals)
        for k in range(CHANNELS // LANES):
            vals = plsc.load_gather(sy, [iota + k * LANES, l_vec])
            plsc.store_scatter(obatch, [slot_vec, iota + (COORDS + k * LANES)],
                               vals)
        plsc.store_scatter(obidx, [slot_vec],
                           jnp.zeros((LANES,), jnp.int32) + p_s,
                           mask=iota == 0)
        slot = slot + 1

        @pl.when(slot == BATCH)
        def _flush():
            pltpu.async_copy(obatch, out_hbm.at[obidx], ssem).wait()
            for k in range(BATCH // LANES):
                obidx[pl.ds(k * LANES, LANES)] = jnp.full(
                    (LANES,), dump, jnp.int32)

        return jnp.where(slot == BATCH, 0, slot)

    def scan_block(grp, sx, sy, slot):
        def scan_vec(j, slot):
            base = pl.multiple_of(j * LANES, LANES)
            e_i = idx_v[pl.ds(base, LANES)]
            e_p = ent_p[pl.ds(base, LANES)]
            m = lax.shift_right_logical(e_i, 8) == grp

            def any_left(c):
                me, _ = c
                return jnp.any(me)

            def pull_one(c):
                me, slot = c
                sel = iota == plsc.all_reduce_ffs(me)
                slot = extract_sample(sx, sy, e_i, e_p, sel, slot)
                return me & jnp.logical_not(sel), slot

            _, slot = lax.while_loop(any_left, pull_one, (m, slot))
            return slot

        return lax.fori_loop(0, nvec, scan_vec, slot)

    @pl.when(nstep > 0)
    def _prologue():
        issue(0, 0)

    def step(r, bufi, slot):
        def active(slot):
            wait_fetch(bufi)

            @pl.when(r + 1 < nstep)
            def _prefetch():
                issue(r + 1, 1 - bufi)

            sx, sy, _ = bufs[bufi]
            return scan_block((lo + 2 * r) >> 1, sx, sy, slot)

        return lax.cond(r < nstep, active, lambda s: s, slot)

    slot = jnp.int32(0)

    def pair(r2, slot):
        slot = step(r2 * 2, 0, slot)
        return step(r2 * 2 + 1, 1, slot)

    slot = lax.fori_loop(0, (NSTEP + 1) // 2, pair, slot)

    def tail(slot):
        pltpu.sync_copy(xtail_hbm, sx0.at[:, pl.ds(0, 128)])
        pltpu.sync_copy(ytail_hbm, sy0.at[:, pl.ds(0, 128)])
        return scan_block(jnp.int32(NFULL >> 1), sx0, sy0, slot)

    slot = lax.cond(is_last, tail, lambda s: s, slot)

    pltpu.async_copy(obatch, out_hbm.at[obidx], ssem).wait()
    del slot


def kernel(source_X, source_Y, indices):
    xtail = jnp.zeros((COORDS, 128), jnp.float32)
    xtail = xtail.at[:, :TAIL_W].set(source_X[TAIL_BASE:].T)
    ytail = jnp.zeros((CHANNELS, 128), jnp.float32)
    ytail = ytail.at[:, :TAIL_W].set(source_Y[TAIL_BASE:].T)
    out2 = _scan_gather(source_X.T, source_Y.T, xtail, ytail, indices)
    return out2[:N, :COORDS], out2[:N, COORDS:COORDS + CHANNELS]


# speedup vs baseline: 3.9821x; 1.0998x over previous
import functools

import jax
import jax.numpy as jnp
from jax import lax
from jax.experimental import pallas as pl
from jax.experimental.pallas import tpu as pltpu
from jax.experimental.pallas import tpu_sc as plsc

SOURCE_SIZE = 1000000
COORDS = 64
CHANNELS = 32
N = 16384

NC, NS = 2, 16
NW = NC * NS
LANES = 16

NTILES = (SOURCE_SIZE + 127) // 128
NFULL = SOURCE_SIZE // 128
TAIL_BASE = NFULL * 128
TAIL_W = SOURCE_SIZE - TAIL_BASE
GW = 256
TPW = 246
NSTEP = TPW // 2
OUT_ROWS = N + NW
BATCH = 128

_mesh = plsc.VectorSubcoreMesh(core_axis_name="c", subcore_axis_name="s")


@functools.partial(
    pl.kernel,
    mesh=_mesh,
    out_type=jax.ShapeDtypeStruct((OUT_ROWS, 128), jnp.float32),
    scratch_types=[
        pltpu.VMEM((N + LANES,), jnp.int32),
        pltpu.VMEM((N,), jnp.int32),
        pltpu.VMEM((COORDS, GW), jnp.float32),
        pltpu.VMEM((COORDS, GW), jnp.float32),
        pltpu.VMEM((COORDS, GW), jnp.float32),
        pltpu.VMEM((CHANNELS, GW), jnp.float32),
        pltpu.VMEM((CHANNELS, GW), jnp.float32),
        pltpu.VMEM((CHANNELS, GW), jnp.float32),
        pltpu.VMEM((BATCH, 128), jnp.float32),
        pltpu.VMEM((BATCH,), jnp.int32),
        pltpu.SemaphoreType.DMA,
        pltpu.SemaphoreType.DMA,
        pltpu.SemaphoreType.DMA,
        pltpu.SemaphoreType.DMA,
    ],
    compiler_params=pltpu.CompilerParams(
        use_tc_tiling_on_sc=True, needs_layout_passes=False),
)
def _scan_gather(xT_hbm, yT_hbm, xtail_hbm, ytail_hbm, idx_hbm, out_hbm,
                 idx_v, ent_p, sx0, sx1, sx2, sy0, sy1, sy2, obatch, obidx,
                 sem0, sem1, sem2, ssem):
    wid = lax.axis_index("s") * NC + lax.axis_index("c")
    lo = wid * TPW
    hi = jnp.minimum(lo + TPW, NFULL)
    nstep = lax.max((hi - lo) // 2, 0)
    is_last = wid == NW - 1
    fhi = jnp.where(is_last, NTILES, hi)
    dump = N + wid
    iota = lax.iota(jnp.int32, LANES)
    bufs = ((sx0, sy0, sem0), (sx1, sy1, sem1), (sx2, sy2, sem2))

    pltpu.sync_copy(idx_hbm, idx_v.at[pl.ds(0, N)])

    def filt(k, cnt):
        iv = idx_v[pl.ds(pl.multiple_of(k * LANES, LANES), LANES)]
        t = lax.shift_right_logical(iv, 7)
        m = (t >= lo) & (t < fhi)
        mi = m.astype(jnp.int32)
        pos = cnt + plsc.cumsum(mi) - 1
        plsc.store_scatter(idx_v, [pos], iv, mask=m)
        plsc.store_scatter(ent_p, [pos], iota + k * LANES, mask=m)
        return cnt + jnp.sum(mi)

    cnt = lax.fori_loop(0, N // LANES, filt, jnp.int32(0))
    plsc.store_scatter(idx_v, [cnt + iota],
                       jnp.full((LANES,), -1, jnp.int32))
    nvec = (cnt + LANES - 1) // LANES

    for k in range(BATCH // LANES):
        obidx[pl.ds(k * LANES, LANES)] = jnp.full((LANES,), dump, jnp.int32)

    def issue(r, bufi):
        sx, sy, sem = bufs[bufi]
        off = pl.multiple_of((lo + 2 * r) * 128, GW)
        pltpu.async_copy(xT_hbm.at[:, pl.ds(off, GW)], sx, sem)
        pltpu.async_copy(yT_hbm.at[:, pl.ds(off, GW)], sy, sem)

    def wait_fetch(bufi):
        sx, sy, sem = bufs[bufi]
        pltpu.make_async_copy(xT_hbm.at[:, pl.ds(0, GW)], sx, sem).wait()
        pltpu.make_async_copy(yT_hbm.at[:, pl.ds(0, GW)], sy, sem).wait()

    def extract_sample(sx, sy, e_i, e_p, sel, slot):
        l_s = jnp.sum(jnp.where(sel, e_i & (GW - 1), 0))
        p_s = jnp.sum(jnp.where(sel, e_p, 0))
        l_vec = jnp.zeros((LANES,), jnp.int32) + l_s
        slot_vec = jnp.zeros((LANES,), jnp.int32) + slot
        for k in range(COORDS // LANES):
            vals = plsc.load_gather(sx, [iota + k * LANES, l_vec])
            plsc.store_scatter(obatch, [slot_vec, iota + k * LANES], vals)
        for k in range(CHANNELS // LANES):
            vals = plsc.load_gather(sy, [iota + k * LANES, l_vec])
            plsc.store_scatter(obatch, [slot_vec, iota + (COORDS + k * LANES)],
                               vals)
        plsc.store_scatter(obidx, [slot_vec],
                           jnp.zeros((LANES,), jnp.int32) + p_s,
                           mask=iota == 0)
        slot = slot + 1

        @pl.when(slot == BATCH)
        def _flush():
            pltpu.async_copy(obatch, out_hbm.at[obidx], ssem).wait()
            for k in range(BATCH // LANES):
                obidx[pl.ds(k * LANES, LANES)] = jnp.full(
                    (LANES,), dump, jnp.int32)

        return jnp.where(slot == BATCH, 0, slot)

    def scan_block(grp, sx, sy, slot):
        def scan_vec(j, slot):
            base = pl.multiple_of(j * LANES, LANES)
            e_i = idx_v[pl.ds(base, LANES)]
            e_p = ent_p[pl.ds(base, LANES)]
            m = lax.shift_right_logical(e_i, 8) == grp

            def any_left(c):
                me, _ = c
                return jnp.any(me)

            def pull_one(c):
                me, slot = c
                sel = iota == plsc.all_reduce_ffs(me)
                slot = extract_sample(sx, sy, e_i, e_p, sel, slot)
                return me & jnp.logical_not(sel), slot

            _, slot = lax.while_loop(any_left, pull_one, (m, slot))
            return slot

        return lax.fori_loop(0, nvec, scan_vec, slot)

    @pl.when(nstep > 0)
    def _prologue0():
        issue(0, 0)

    @pl.when(nstep > 1)
    def _prologue1():
        issue(1, 1)

    def step(r, bufi, slot):
        def active(slot):
            wait_fetch(bufi)

            @pl.when(r + 2 < nstep)
            def _prefetch():
                issue(r + 2, (bufi + 2) % 3)

            sx, sy, _ = bufs[bufi]
            return scan_block((lo + 2 * r) >> 1, sx, sy, slot)

        return lax.cond(r < nstep, active, lambda s: s, slot)

    slot = jnp.int32(0)

    def triple(r3, slot):
        slot = step(r3 * 3, 0, slot)
        slot = step(r3 * 3 + 1, 1, slot)
        return step(r3 * 3 + 2, 2, slot)

    slot = lax.fori_loop(0, NSTEP // 3, triple, slot)

    def tail(slot):
        pltpu.sync_copy(xtail_hbm, sx0.at[:, pl.ds(0, 128)])
        pltpu.sync_copy(ytail_hbm, sy0.at[:, pl.ds(0, 128)])
        return scan_block(jnp.int32(NFULL >> 1), sx0, sy0, slot)

    slot = lax.cond(is_last, tail, lambda s: s, slot)

    pltpu.async_copy(obatch, out_hbm.at[obidx], ssem).wait()
    del slot


def kernel(source_X, source_Y, indices):
    xtail = jnp.zeros((COORDS, 128), jnp.float32)
    xtail = xtail.at[:, :TAIL_W].set(source_X[TAIL_BASE:].T)
    ytail = jnp.zeros((CHANNELS, 128), jnp.float32)
    ytail = ytail.at[:, :TAIL_W].set(source_Y[TAIL_BASE:].T)
    out2 = _scan_gather(source_X.T, source_Y.T, xtail, ytail, indices)
    return out2[:N, :COORDS], out2[:N, COORDS:COORDS + CHANNELS]
